# Initial kernel scaffold; baseline (speedup 1.0000x reference)
import jax
import jax.numpy as jnp
from jax.experimental import pallas as pl


def kernel(rois, mrcnn_class, mrcnn_bbox, image_meta):
    raise NotImplementedError("write your pallas kernel here")


# speedup vs baseline: 2.4968x; 2.4968x over previous
import jax
import jax.numpy as jnp
from jax import lax
from jax.experimental import pallas as pl
from jax.experimental.pallas import tpu as pltpu
from jax.experimental.pallas import tpu_sc as plsc

_B, _N, _C = 8, 2000, 81
_QPAD = 512
_NPAD = 4 * _QPAD
_NROWS = _B * _N + 48
_MAX_INST = 100
_MIN_CONF = 0.7
_NMS_THR = 0.3
_VROWS = _B * _N * _C * 4 // 128

_OY1, _OX1, _OY2, _OX2, _AREA, _SM, _RY1, _RX1, _RY2, _RX2, _CLS, _SCORE = range(12)
_NCOL = 12

_f32 = jnp.float32
_i32 = jnp.int32


def _splat_f(x):
    return jnp.full((16,), x, _f32)


def _splat_i(x):
    return jnp.full((16,), x, _i32)


def _det_body(probs_hbm, rois_hbm, bbxt_hbm, win_hbm, out_hbm,
              probs_v, rois_v, clsq_v, offq_v, idx_v, dstage_v, colq_v,
              win_v, shared_v, full_v, sel_v, val_v, out_v, sem):
    cidx = lax.axis_index("c")
    sidx = lax.axis_index("s")
    img_local = sidx // 4
    quarter = sidx % 4
    img = cidx * 4 + img_local
    base = img * _N + quarter * _QPAD

    lanes = lax.iota(_i32, 16)

    pltpu.sync_copy(probs_hbm.at[pl.ds(base * _C, _QPAD * _C)], probs_v)
    pltpu.sync_copy(rois_hbm.at[pl.ds(base * 4, _QPAD * 4)], rois_v)
    pltpu.sync_copy(win_hbm, win_v)
    wy1v = plsc.load_gather(win_v, [_splat_i(img * 16 + 0)])
    wx1v = plsc.load_gather(win_v, [_splat_i(img * 16 + 1)])
    wy2v = plsc.load_gather(win_v, [_splat_i(img * 16 + 2)])
    wx2v = plsc.load_gather(win_v, [_splat_i(img * 16 + 3)])

    def group1(g, _):
        pos = lanes + g * 16
        pbase = pos * _C

        def cls_body(c, carry):
            best, bidx = carry
            vals = plsc.load_gather(probs_v, [pbase + c])
            upd = vals > best
            return (jnp.where(upd, vals, best), jnp.where(upd, c, bidx))

        best, bidx = lax.fori_loop(
            0, _C, cls_body, (_splat_f(-1.0), _splat_i(0)))
        gsrc = jnp.minimum(base + pos, _B * _N - 1)
        flat = gsrc * (_C * 4) + bidx * 4
        row = lax.shift_right_logical(flat, 7)
        off = lax.bitwise_and(flat, 127)
        plsc.store_scatter(idx_v, [pos], row)
        plsc.store_scatter(clsq_v, [pos], bidx)
        plsc.store_scatter(offq_v, [pos], off)
        plsc.store_scatter(colq_v, [_splat_i(_SCORE * _QPAD) + pos], best)
        return 0

    lax.fori_loop(0, _QPAD // 16, group1, 0)

    for h in range(2):
        copies = [
            pltpu.async_copy(
                bbxt_hbm.at[idx_v.at[pl.ds((2 * h + j) * 128, 128)]],
                dstage_v.at[pl.ds(j * 128, 128)], sem)
            for j in range(2)
        ]
        for c in copies:
            c.wait()

        def group2(g, _, h=h):
            pos = lanes + g * 16
            lp = pos - h * 256
            offv = plsc.load_gather(offq_v, [pos])
            d0 = plsc.load_gather(dstage_v, [lp, offv]) * 0.1
            d1 = plsc.load_gather(dstage_v, [lp, offv + 1]) * 0.1
            d2 = plsc.load_gather(dstage_v, [lp, offv + 2]) * 0.2
            d3 = plsc.load_gather(dstage_v, [lp, offv + 3]) * 0.2
            rbase = pos * 4
            y1 = plsc.load_gather(rois_v, [rbase])
            x1 = plsc.load_gather(rois_v, [rbase + 1])
            y2 = plsc.load_gather(rois_v, [rbase + 2])
            x2 = plsc.load_gather(rois_v, [rbase + 3])
            cls_i = plsc.load_gather(clsq_v, [pos])
            score = plsc.load_gather(colq_v, [_splat_i(_SCORE * _QPAD) + pos])

            hh = y2 - y1
            ww = x2 - x1
            cy = y1 + 0.5 * hh
            cx = x1 + 0.5 * ww
            cy = cy + d0 * hh
            cx = cx + d1 * ww
            hh = hh * jnp.exp(d2)
            ww = ww * jnp.exp(d3)
            ry1 = cy - 0.5 * hh
            rx1 = cx - 0.5 * ww
            ry2 = ry1 + hh
            rx2 = rx1 + ww
            ry1 = jnp.minimum(jnp.maximum(ry1, wy1v), wy2v)
            rx1 = jnp.minimum(jnp.maximum(rx1, wx1v), wx2v)
            ry2 = jnp.minimum(jnp.maximum(ry2, wy1v), wy2v)
            rx2 = jnp.minimum(jnp.maximum(rx2, wx1v), wx2v)

            keep = ((cls_i > 0) & (score >= _MIN_CONF)
                    & (quarter * _QPAD + pos < _N))
            sm = jnp.where(keep, score, -1.0)
            clsf = cls_i.astype(_f32)
            off = clsf * 2.0
            oy1 = ry1 + off
            ox1 = rx1 + off
            oy2 = ry2 + off
            ox2 = rx2 + off
            area = (oy2 - oy1) * (ox2 - ox1)

            for col, x in ((_OY1, oy1), (_OX1, ox1), (_OY2, oy2), (_OX2, ox2),
                           (_AREA, area), (_SM, sm), (_RY1, ry1), (_RX1, rx1),
                           (_RY2, ry2), (_RX2, rx2), (_CLS, clsf)):
                plsc.store_scatter(colq_v, [_splat_i(col * _QPAD) + pos], x)
            return 0

        lax.fori_loop(h * 16, (h + 1) * 16, group2, 0)

    for c in range(_NCOL):
        pltpu.sync_copy(colq_v.at[pl.ds(c * _QPAD, _QPAD)],
                        shared_v.at[c, img_local, pl.ds(quarter * _QPAD, _QPAD)])
    plsc.subcore_barrier()

    @pl.when(quarter == 0)
    def _nms():
        for c in range(_NCOL):
            pltpu.sync_copy(shared_v.at[c, img_local],
                            full_v.at[pl.ds(c * _NPAD, _NPAD)])

        def step(t, carry):
            by1, bx1, by2, bx2, ba = carry

            def chunk(ch, acc):
                best, bchunk = acc
                posv = lanes + ch * 16
                cy1 = plsc.load_gather(full_v, [_splat_i(_OY1 * _NPAD) + posv])
                cx1 = plsc.load_gather(full_v, [_splat_i(_OX1 * _NPAD) + posv])
                cy2 = plsc.load_gather(full_v, [_splat_i(_OY2 * _NPAD) + posv])
                cx2 = plsc.load_gather(full_v, [_splat_i(_OX2 * _NPAD) + posv])
                ca = plsc.load_gather(full_v, [_splat_i(_AREA * _NPAD) + posv])
                s = plsc.load_gather(full_v, [_splat_i(_SM * _NPAD) + posv])
                yy1 = jnp.maximum(by1, cy1)
                xx1 = jnp.maximum(bx1, cx1)
                yy2 = jnp.minimum(by2, cy2)
                xx2 = jnp.minimum(bx2, cx2)
                inter = (jnp.maximum(yy2 - yy1, 0.0)
                         * jnp.maximum(xx2 - xx1, 0.0))
                union = ba + ca - inter
                iou = inter / jnp.maximum(union, 1e-10)
                s = jnp.where(iou >= _NMS_THR, -1.0, s)
                plsc.store_scatter(full_v, [_splat_i(_SM * _NPAD) + posv], s)
                upd = s > best
                best = jnp.where(upd, s, best)
                bchunk = jnp.where(upd, ch, bchunk)
                return (best, bchunk)

            best, bchunk = lax.fori_loop(
                0, _NPAD // 16, chunk, (_splat_f(-2.0), _splat_i(0)))
            m = jnp.max(best)
            gidx = bchunk * 16 + lanes
            idx = jnp.min(jnp.where(best == m, gidx, _NPAD))
            val = jnp.where(m > 0.0, _f32(1.0), _f32(0.0))
            lane0 = lanes == 0
            plsc.store_scatter(sel_v, [_splat_i(t)], _splat_i(idx), mask=lane0)
            plsc.store_scatter(val_v, [_splat_i(t)], _splat_f(val), mask=lane0)
            iv = _splat_i(idx)
            return (plsc.load_gather(full_v, [_splat_i(_OY1 * _NPAD) + iv]),
                    plsc.load_gather(full_v, [_splat_i(_OX1 * _NPAD) + iv]),
                    plsc.load_gather(full_v, [_splat_i(_OY2 * _NPAD) + iv]),
                    plsc.load_gather(full_v, [_splat_i(_OX2 * _NPAD) + iv]),
                    plsc.load_gather(full_v, [_splat_i(_AREA * _NPAD) + iv]))

        far = _splat_f(-1.0e9)
        lax.fori_loop(0, _MAX_INST, step,
                      (far, far, far, far, _splat_f(0.0)))

        def outg(g, _):
            t16 = lanes + g * 16
            tc = jnp.minimum(t16, _MAX_INST - 1)
            sidxs = plsc.load_gather(sel_v, [tc])
            v = plsc.load_gather(val_v, [tc]) > 0.0
            for k, col in enumerate((_RY1, _RX1, _RY2, _RX2, _CLS, _SCORE)):
                x = plsc.load_gather(full_v, [_splat_i(col * _NPAD) + sidxs])
                x = jnp.where(v, x, 0.0)
                plsc.store_scatter(out_v, [t16 * 8 + k], x)
            return 0

        lax.fori_loop(0, 7, outg, 0)
        pltpu.sync_copy(out_v, out_hbm.at[img])


def kernel(rois, mrcnn_class, mrcnn_bbox, image_meta):
    probs2 = jnp.pad(mrcnn_class.reshape(_B * _N * _C),
                     (0, (_NROWS - _B * _N) * _C))
    rois2 = jnp.pad(rois.reshape(_B * _N * 4),
                    (0, (_NROWS - _B * _N) * 4))
    bbxt = mrcnn_bbox.reshape(_VROWS, 128)
    image_shape = image_meta[0, 4:7]
    h, w = image_shape[0], image_shape[1]
    scale = jnp.stack([h, w, h, w]) - 1.0
    shift = jnp.array([0.0, 0.0, 1.0, 1.0], dtype=_f32)
    windows = (image_meta[:, 7:11] - shift) / scale
    winflat = jnp.pad(windows, ((0, 0), (0, 12))).reshape(128)

    mesh = plsc.VectorSubcoreMesh(core_axis_name="c", subcore_axis_name="s")
    det = pl.kernel(
        _det_body,
        mesh=mesh,
        compiler_params=pltpu.CompilerParams(needs_layout_passes=False),
        out_type=jax.ShapeDtypeStruct((_B, 1024), _f32),
        scratch_types=[
            pltpu.VMEM((_QPAD * _C,), _f32),
            pltpu.VMEM((_QPAD * 4,), _f32),
            pltpu.VMEM((_QPAD,), _i32),
            pltpu.VMEM((_QPAD,), _i32),
            pltpu.VMEM((_QPAD,), _i32),
            pltpu.VMEM((256, 128), _f32),
            pltpu.VMEM((_NCOL * _QPAD,), _f32),
            pltpu.VMEM((128,), _f32),
            pltpu.VMEM_SHARED((_NCOL, 4, _NPAD), _f32),
            pltpu.VMEM((_NCOL * _NPAD,), _f32),
            pltpu.VMEM((128,), _i32),
            pltpu.VMEM((128,), _f32),
            pltpu.VMEM((1024,), _f32),
            pltpu.SemaphoreType.DMA,
        ],
    )
    out = det(probs2, rois2, bbxt, winflat)
    return out.reshape(_B, 128, 8)[:, :_MAX_INST, :6]


# speedup vs baseline: 2.7609x; 1.1058x over previous
import jax
import jax.numpy as jnp
from jax import lax
from jax.experimental import pallas as pl
from jax.experimental.pallas import tpu as pltpu
from jax.experimental.pallas import tpu_sc as plsc

_B, _N, _C = 8, 2000, 81
_QPAD = 512
_NPAD = 4 * _QPAD
_NROWS = _B * _N + 48
_MAX_INST = 100
_MIN_CONF = 0.7
_NMS_THR = 0.3
_VROWS = _B * _N * _C * 4 // 128

_OY1, _OX1, _OY2, _OX2, _AREA, _SM, _RY1, _RX1, _RY2, _RX2, _CLS, _SCORE = range(12)
_NCOL = 12

_f32 = jnp.float32
_i32 = jnp.int32


def _splat_f(x):
    return jnp.full((16,), x, _f32)


def _splat_i(x):
    return jnp.full((16,), x, _i32)


def _det_body(probs_hbm, rois_hbm, bbxt_hbm, win_hbm, out_hbm,
              probs_v, rois_v, clsq_v, offq_v, idx_v, dstage_v, colq_v,
              win_v, shared_v, full_v, sel_v, val_v, out_v, sem):
    cidx = lax.axis_index("c")
    sidx = lax.axis_index("s")
    img_local = sidx // 4
    quarter = sidx % 4
    img = cidx * 4 + img_local
    base = img * _N + quarter * _QPAD

    lanes = lax.iota(_i32, 16)

    pltpu.sync_copy(probs_hbm.at[pl.ds(base * _C, _QPAD * _C)], probs_v)
    pltpu.sync_copy(rois_hbm.at[pl.ds(base * 4, _QPAD * 4)], rois_v)
    pltpu.sync_copy(win_hbm, win_v)
    wy1v = plsc.load_gather(win_v, [_splat_i(img * 16 + 0)])
    wx1v = plsc.load_gather(win_v, [_splat_i(img * 16 + 1)])
    wy2v = plsc.load_gather(win_v, [_splat_i(img * 16 + 2)])
    wx2v = plsc.load_gather(win_v, [_splat_i(img * 16 + 3)])

    @plsc.parallel_loop(0, _QPAD // 16, unroll=2)
    def group1(g):
        pos = lanes + g * 16
        pbase = pos * _C

        @plsc.parallel_loop(0, _C, unroll=3,
                            carry=(_splat_f(-1.0), _splat_i(0)))
        def cls_loop(c, carry):
            best, bidx = carry
            vals = plsc.load_gather(probs_v, [pbase + c])
            upd = vals > best
            return (jnp.where(upd, vals, best), jnp.where(upd, c, bidx))

        best, bidx = cls_loop
        gsrc = jnp.minimum(base + pos, _B * _N - 1)
        flat = gsrc * (_C * 4) + bidx * 4
        row = lax.shift_right_logical(flat, 7)
        off = lax.bitwise_and(flat, 127)
        o = g * 16
        idx_v[pl.ds(o, 16)] = row
        clsq_v[pl.ds(o, 16)] = bidx
        offq_v[pl.ds(o, 16)] = off
        colq_v[pl.ds(_SCORE * _QPAD + o, 16)] = best

    for h in range(2):
        copies = [
            pltpu.async_copy(
                bbxt_hbm.at[idx_v.at[pl.ds((2 * h + j) * 128, 128)]],
                dstage_v.at[pl.ds(j * 128, 128)], sem)
            for j in range(2)
        ]
        for c in copies:
            c.wait()

        @plsc.parallel_loop(h * 16, (h + 1) * 16, unroll=2)
        def group2(g, h=h):
            pos = lanes + g * 16
            o = g * 16
            lp = pos - h * 256
            offv = offq_v[pl.ds(o, 16)]
            d0 = plsc.load_gather(dstage_v, [lp, offv]) * 0.1
            d1 = plsc.load_gather(dstage_v, [lp, offv + 1]) * 0.1
            d2 = plsc.load_gather(dstage_v, [lp, offv + 2]) * 0.2
            d3 = plsc.load_gather(dstage_v, [lp, offv + 3]) * 0.2
            rbase = pos * 4
            y1 = plsc.load_gather(rois_v, [rbase])
            x1 = plsc.load_gather(rois_v, [rbase + 1])
            y2 = plsc.load_gather(rois_v, [rbase + 2])
            x2 = plsc.load_gather(rois_v, [rbase + 3])
            cls_i = clsq_v[pl.ds(o, 16)]
            score = colq_v[pl.ds(_SCORE * _QPAD + o, 16)]

            hh = y2 - y1
            ww = x2 - x1
            cy = y1 + 0.5 * hh
            cx = x1 + 0.5 * ww
            cy = cy + d0 * hh
            cx = cx + d1 * ww
            hh = hh * jnp.exp(d2)
            ww = ww * jnp.exp(d3)
            ry1 = cy - 0.5 * hh
            rx1 = cx - 0.5 * ww
            ry2 = ry1 + hh
            rx2 = rx1 + ww
            ry1 = jnp.minimum(jnp.maximum(ry1, wy1v), wy2v)
            rx1 = jnp.minimum(jnp.maximum(rx1, wx1v), wx2v)
            ry2 = jnp.minimum(jnp.maximum(ry2, wy1v), wy2v)
            rx2 = jnp.minimum(jnp.maximum(rx2, wx1v), wx2v)

            keep = ((cls_i > 0) & (score >= _MIN_CONF)
                    & (quarter * _QPAD + pos < _N))
            sm = jnp.where(keep, score, -1.0)
            clsf = cls_i.astype(_f32)
            off = clsf * 2.0
            oy1 = ry1 + off
            ox1 = rx1 + off
            oy2 = ry2 + off
            ox2 = rx2 + off
            area = (oy2 - oy1) * (ox2 - ox1)

            for col, x in ((_OY1, oy1), (_OX1, ox1), (_OY2, oy2), (_OX2, ox2),
                           (_AREA, area), (_SM, sm), (_RY1, ry1), (_RX1, rx1),
                           (_RY2, ry2), (_RX2, rx2), (_CLS, clsf)):
                colq_v[pl.ds(col * _QPAD + o, 16)] = x

    for c in range(_NCOL):
        pltpu.sync_copy(colq_v.at[pl.ds(c * _QPAD, _QPAD)],
                        shared_v.at[c, img_local, pl.ds(quarter * _QPAD, _QPAD)])
    plsc.subcore_barrier()

    @pl.when(quarter == 0)
    def _nms():
        for c in range(_NCOL):
            pltpu.sync_copy(shared_v.at[c, img_local],
                            full_v.at[pl.ds(c * _NPAD, _NPAD)])

        def step(t, carry):
            by1, bx1, by2, bx2, ba = carry

            @plsc.parallel_loop(0, _NPAD // 16, unroll=4,
                                carry=(_splat_f(-2.0), _splat_i(0)))
            def chunk(ch, acc):
                best, bchunk = acc
                o = ch * 16
                cy1 = full_v[pl.ds(_OY1 * _NPAD + o, 16)]
                cx1 = full_v[pl.ds(_OX1 * _NPAD + o, 16)]
                cy2 = full_v[pl.ds(_OY2 * _NPAD + o, 16)]
                cx2 = full_v[pl.ds(_OX2 * _NPAD + o, 16)]
                ca = full_v[pl.ds(_AREA * _NPAD + o, 16)]
                s = full_v[pl.ds(_SM * _NPAD + o, 16)]
                yy1 = jnp.maximum(by1, cy1)
                xx1 = jnp.maximum(bx1, cx1)
                yy2 = jnp.minimum(by2, cy2)
                xx2 = jnp.minimum(bx2, cx2)
                inter = (jnp.maximum(yy2 - yy1, 0.0)
                         * jnp.maximum(xx2 - xx1, 0.0))
                union = ba + ca - inter
                iou = inter / jnp.maximum(union, 1e-10)
                s = jnp.where(iou >= _NMS_THR, -1.0, s)
                full_v[pl.ds(_SM * _NPAD + o, 16)] = s
                upd = s > best
                best = jnp.where(upd, s, best)
                bchunk = jnp.where(upd, ch, bchunk)
                return (best, bchunk)

            best, bchunk = chunk
            m = jnp.max(best)
            gidx = bchunk * 16 + lanes
            idx = jnp.min(jnp.where(best == m, gidx, _NPAD))
            val = jnp.where(m > 0.0, _f32(1.0), _f32(0.0))
            lane0 = lanes == 0
            plsc.store_scatter(sel_v, [_splat_i(t)], _splat_i(idx), mask=lane0)
            plsc.store_scatter(val_v, [_splat_i(t)], _splat_f(val), mask=lane0)
            iv = _splat_i(idx)
            return (plsc.load_gather(full_v, [_splat_i(_OY1 * _NPAD) + iv]),
                    plsc.load_gather(full_v, [_splat_i(_OX1 * _NPAD) + iv]),
                    plsc.load_gather(full_v, [_splat_i(_OY2 * _NPAD) + iv]),
                    plsc.load_gather(full_v, [_splat_i(_OX2 * _NPAD) + iv]),
                    plsc.load_gather(full_v, [_splat_i(_AREA * _NPAD) + iv]))

        far = _splat_f(-1.0e9)
        lax.fori_loop(0, _MAX_INST, step,
                      (far, far, far, far, _splat_f(0.0)))

        def outg(g, _):
            t16 = lanes + g * 16
            tc = jnp.minimum(t16, _MAX_INST - 1)
            sidxs = plsc.load_gather(sel_v, [tc])
            v = plsc.load_gather(val_v, [tc]) > 0.0
            for k, col in enumerate((_RY1, _RX1, _RY2, _RX2, _CLS, _SCORE)):
                x = plsc.load_gather(full_v, [_splat_i(col * _NPAD) + sidxs])
                x = jnp.where(v, x, 0.0)
                plsc.store_scatter(out_v, [t16 * 8 + k], x)
            return 0

        lax.fori_loop(0, 7, outg, 0)
        pltpu.sync_copy(out_v, out_hbm.at[img])


def kernel(rois, mrcnn_class, mrcnn_bbox, image_meta):
    probs2 = jnp.pad(mrcnn_class.reshape(_B * _N * _C),
                     (0, (_NROWS - _B * _N) * _C))
    rois2 = jnp.pad(rois.reshape(_B * _N * 4),
                    (0, (_NROWS - _B * _N) * 4))
    bbxt = mrcnn_bbox.reshape(_VROWS, 128)
    image_shape = image_meta[0, 4:7]
    h, w = image_shape[0], image_shape[1]
    scale = jnp.stack([h, w, h, w]) - 1.0
    shift = jnp.array([0.0, 0.0, 1.0, 1.0], dtype=_f32)
    windows = (image_meta[:, 7:11] - shift) / scale
    winflat = jnp.pad(windows, ((0, 0), (0, 12))).reshape(128)

    mesh = plsc.VectorSubcoreMesh(core_axis_name="c", subcore_axis_name="s")
    det = pl.kernel(
        _det_body,
        mesh=mesh,
        compiler_params=pltpu.CompilerParams(needs_layout_passes=False),
        out_type=jax.ShapeDtypeStruct((_B, 1024), _f32),
        scratch_types=[
            pltpu.VMEM((_QPAD * _C,), _f32),
            pltpu.VMEM((_QPAD * 4,), _f32),
            pltpu.VMEM((_QPAD,), _i32),
            pltpu.VMEM((_QPAD,), _i32),
            pltpu.VMEM((_QPAD,), _i32),
            pltpu.VMEM((256, 128), _f32),
            pltpu.VMEM((_NCOL * _QPAD,), _f32),
            pltpu.VMEM((128,), _f32),
            pltpu.VMEM_SHARED((_NCOL, 4, _NPAD), _f32),
            pltpu.VMEM((_NCOL * _NPAD,), _f32),
            pltpu.VMEM((128,), _i32),
            pltpu.VMEM((128,), _f32),
            pltpu.VMEM((1024,), _f32),
            pltpu.SemaphoreType.DMA,
        ],
    )
    out = det(probs2, rois2, bbxt, winflat)
    return out.reshape(_B, 128, 8)[:, :_MAX_INST, :6]


# speedup vs baseline: 2.7666x; 1.0021x over previous
import jax
import jax.numpy as jnp
from jax import lax
from jax.experimental import pallas as pl
from jax.experimental.pallas import tpu as pltpu
from jax.experimental.pallas import tpu_sc as plsc

_B, _N, _C = 8, 2000, 81
_QPAD = 512
_NPAD = 4 * _QPAD
_NROWS = _B * _N + 48
_MAX_INST = 100
_MIN_CONF = 0.7
_NMS_THR = 0.3
_VROWS = _B * _N * _C * 4 // 128

_OY1, _OX1, _OY2, _OX2, _AREA, _SM, _RY1, _RX1, _RY2, _RX2, _CLS, _SCORE = range(12)
_NCOL = 12

_f32 = jnp.float32
_i32 = jnp.int32


def _splat_f(x):
    return jnp.full((16,), x, _f32)


def _splat_i(x):
    return jnp.full((16,), x, _i32)


def _det_body(probs_hbm, rois_hbm, bbxt_hbm, win_hbm, out_hbm,
              probs_v, rois_v, clsq_v, offq_v, idx_v, dstage_v, colq_v,
              win_v, shared_v, full_v, sel_v, val_v, out_v, sem):
    cidx = lax.axis_index("c")
    sidx = lax.axis_index("s")
    img_local = sidx // 4
    quarter = sidx % 4
    img = cidx * 4 + img_local
    base = img * _N + quarter * _QPAD

    lanes = lax.iota(_i32, 16)

    pltpu.sync_copy(probs_hbm.at[pl.ds(base * _C, _QPAD * _C)], probs_v)
    pltpu.sync_copy(rois_hbm.at[pl.ds(base * 4, _QPAD * 4)], rois_v)
    pltpu.sync_copy(win_hbm, win_v)
    wy1v = plsc.load_gather(win_v, [_splat_i(img * 16 + 0)])
    wx1v = plsc.load_gather(win_v, [_splat_i(img * 16 + 1)])
    wy2v = plsc.load_gather(win_v, [_splat_i(img * 16 + 2)])
    wx2v = plsc.load_gather(win_v, [_splat_i(img * 16 + 3)])

    @plsc.parallel_loop(0, _QPAD // 16, unroll=2)
    def group1(g):
        pos = lanes + g * 16
        pbase = pos * _C

        @plsc.parallel_loop(0, _C, unroll=3,
                            carry=(_splat_f(-1.0), _splat_i(0)))
        def cls_loop(c, carry):
            best, bidx = carry
            vals = plsc.load_gather(probs_v, [pbase + c])
            upd = vals > best
            return (jnp.where(upd, vals, best), jnp.where(upd, c, bidx))

        best, bidx = cls_loop
        gsrc = jnp.minimum(base + pos, _B * _N - 1)
        flat = gsrc * (_C * 4) + bidx * 4
        row = lax.shift_right_logical(flat, 7)
        off = lax.bitwise_and(flat, 127)
        o = g * 16
        idx_v[pl.ds(o, 16)] = row
        clsq_v[pl.ds(o, 16)] = bidx
        offq_v[pl.ds(o, 16)] = off
        colq_v[pl.ds(_SCORE * _QPAD + o, 16)] = best

    for h in range(2):
        copies = [
            pltpu.async_copy(
                bbxt_hbm.at[idx_v.at[pl.ds((2 * h + j) * 128, 128)]],
                dstage_v.at[pl.ds(j * 128, 128)], sem)
            for j in range(2)
        ]
        for c in copies:
            c.wait()

        @plsc.parallel_loop(h * 16, (h + 1) * 16, unroll=2)
        def group2(g, h=h):
            pos = lanes + g * 16
            o = g * 16
            lp = pos - h * 256
            offv = offq_v[pl.ds(o, 16)]
            d0 = plsc.load_gather(dstage_v, [lp, offv]) * 0.1
            d1 = plsc.load_gather(dstage_v, [lp, offv + 1]) * 0.1
            d2 = plsc.load_gather(dstage_v, [lp, offv + 2]) * 0.2
            d3 = plsc.load_gather(dstage_v, [lp, offv + 3]) * 0.2
            rbase = pos * 4
            y1 = plsc.load_gather(rois_v, [rbase])
            x1 = plsc.load_gather(rois_v, [rbase + 1])
            y2 = plsc.load_gather(rois_v, [rbase + 2])
            x2 = plsc.load_gather(rois_v, [rbase + 3])
            cls_i = clsq_v[pl.ds(o, 16)]
            score = colq_v[pl.ds(_SCORE * _QPAD + o, 16)]

            hh = y2 - y1
            ww = x2 - x1
            cy = y1 + 0.5 * hh
            cx = x1 + 0.5 * ww
            cy = cy + d0 * hh
            cx = cx + d1 * ww
            hh = hh * jnp.exp(d2)
            ww = ww * jnp.exp(d3)
            ry1 = cy - 0.5 * hh
            rx1 = cx - 0.5 * ww
            ry2 = ry1 + hh
            rx2 = rx1 + ww
            ry1 = jnp.minimum(jnp.maximum(ry1, wy1v), wy2v)
            rx1 = jnp.minimum(jnp.maximum(rx1, wx1v), wx2v)
            ry2 = jnp.minimum(jnp.maximum(ry2, wy1v), wy2v)
            rx2 = jnp.minimum(jnp.maximum(rx2, wx1v), wx2v)

            keep = ((cls_i > 0) & (score >= _MIN_CONF)
                    & (quarter * _QPAD + pos < _N))
            sm = jnp.where(keep, score, -1.0)
            clsf = cls_i.astype(_f32)
            off = clsf * 2.0
            oy1 = ry1 + off
            ox1 = rx1 + off
            oy2 = ry2 + off
            ox2 = rx2 + off
            area = (oy2 - oy1) * (ox2 - ox1)

            for col, x in ((_OY1, oy1), (_OX1, ox1), (_OY2, oy2), (_OX2, ox2),
                           (_AREA, area), (_SM, sm), (_RY1, ry1), (_RX1, rx1),
                           (_RY2, ry2), (_RX2, rx2), (_CLS, clsf)):
                colq_v[pl.ds(col * _QPAD + o, 16)] = x

    for c in range(_NCOL):
        pltpu.sync_copy(colq_v.at[pl.ds(c * _QPAD, _QPAD)],
                        shared_v.at[c, img_local, pl.ds(quarter * _QPAD, _QPAD)])
    plsc.subcore_barrier()

    @pl.when(quarter == 0)
    def _nms():
        for c in range(_NCOL):
            pltpu.sync_copy(shared_v.at[c, img_local],
                            full_v.at[pl.ds(c * _NPAD, _NPAD)])

        def step(t, carry):
            by1, bx1, by2, bx2, ba = carry

            @plsc.parallel_loop(0, _NPAD // 16, unroll=4,
                                carry=(_splat_f(-2.0), _splat_i(0)))
            def chunk(ch, acc):
                best, bchunk = acc
                o = ch * 16
                cy1 = full_v[pl.ds(_OY1 * _NPAD + o, 16)]
                cx1 = full_v[pl.ds(_OX1 * _NPAD + o, 16)]
                cy2 = full_v[pl.ds(_OY2 * _NPAD + o, 16)]
                cx2 = full_v[pl.ds(_OX2 * _NPAD + o, 16)]
                ca = full_v[pl.ds(_AREA * _NPAD + o, 16)]
                s = full_v[pl.ds(_SM * _NPAD + o, 16)]
                yy1 = jnp.maximum(by1, cy1)
                xx1 = jnp.maximum(bx1, cx1)
                yy2 = jnp.minimum(by2, cy2)
                xx2 = jnp.minimum(bx2, cx2)
                inter = (jnp.maximum(yy2 - yy1, 0.0)
                         * jnp.maximum(xx2 - xx1, 0.0))
                union = ba + ca - inter
                s = jnp.where(inter >= _NMS_THR * jnp.maximum(union, 1e-10),
                              -1.0, s)
                full_v[pl.ds(_SM * _NPAD + o, 16)] = s
                upd = s > best
                best = jnp.where(upd, s, best)
                bchunk = jnp.where(upd, ch, bchunk)
                return (best, bchunk)

            best, bchunk = chunk
            m = jnp.max(best)
            gidx = bchunk * 16 + lanes
            idx = jnp.min(jnp.where(best == m, gidx, _NPAD))
            val = jnp.where(m > 0.0, _f32(1.0), _f32(0.0))
            lane0 = lanes == 0
            plsc.store_scatter(sel_v, [_splat_i(t)], _splat_i(idx), mask=lane0)
            plsc.store_scatter(val_v, [_splat_i(t)], _splat_f(val), mask=lane0)
            iv = _splat_i(idx)
            return (plsc.load_gather(full_v, [_splat_i(_OY1 * _NPAD) + iv]),
                    plsc.load_gather(full_v, [_splat_i(_OX1 * _NPAD) + iv]),
                    plsc.load_gather(full_v, [_splat_i(_OY2 * _NPAD) + iv]),
                    plsc.load_gather(full_v, [_splat_i(_OX2 * _NPAD) + iv]),
                    plsc.load_gather(full_v, [_splat_i(_AREA * _NPAD) + iv]))

        far = _splat_f(-1.0e9)
        lax.fori_loop(0, _MAX_INST, step,
                      (far, far, far, far, _splat_f(0.0)))

        def outg(g, _):
            t16 = lanes + g * 16
            tc = jnp.minimum(t16, _MAX_INST - 1)
            sidxs = plsc.load_gather(sel_v, [tc])
            v = plsc.load_gather(val_v, [tc]) > 0.0
            for k, col in enumerate((_RY1, _RX1, _RY2, _RX2, _CLS, _SCORE)):
                x = plsc.load_gather(full_v, [_splat_i(col * _NPAD) + sidxs])
                x = jnp.where(v, x, 0.0)
                plsc.store_scatter(out_v, [t16 * 8 + k], x)
            return 0

        lax.fori_loop(0, 7, outg, 0)
        pltpu.sync_copy(out_v, out_hbm.at[img])


def kernel(rois, mrcnn_class, mrcnn_bbox, image_meta):
    probs2 = jnp.pad(mrcnn_class.reshape(_B * _N * _C),
                     (0, (_NROWS - _B * _N) * _C))
    rois2 = jnp.pad(rois.reshape(_B * _N * 4),
                    (0, (_NROWS - _B * _N) * 4))
    bbxt = mrcnn_bbox.reshape(_VROWS, 128)
    image_shape = image_meta[0, 4:7]
    h, w = image_shape[0], image_shape[1]
    scale = jnp.stack([h, w, h, w]) - 1.0
    shift = jnp.array([0.0, 0.0, 1.0, 1.0], dtype=_f32)
    windows = (image_meta[:, 7:11] - shift) / scale
    winflat = jnp.pad(windows, ((0, 0), (0, 12))).reshape(128)

    mesh = plsc.VectorSubcoreMesh(core_axis_name="c", subcore_axis_name="s")
    det = pl.kernel(
        _det_body,
        mesh=mesh,
        compiler_params=pltpu.CompilerParams(needs_layout_passes=False),
        out_type=jax.ShapeDtypeStruct((_B, 1024), _f32),
        scratch_types=[
            pltpu.VMEM((_QPAD * _C,), _f32),
            pltpu.VMEM((_QPAD * 4,), _f32),
            pltpu.VMEM((_QPAD,), _i32),
            pltpu.VMEM((_QPAD,), _i32),
            pltpu.VMEM((_QPAD,), _i32),
            pltpu.VMEM((256, 128), _f32),
            pltpu.VMEM((_NCOL * _QPAD,), _f32),
            pltpu.VMEM((128,), _f32),
            pltpu.VMEM_SHARED((_NCOL, 4, _NPAD), _f32),
            pltpu.VMEM((_NCOL * _NPAD,), _f32),
            pltpu.VMEM((128,), _i32),
            pltpu.VMEM((128,), _f32),
            pltpu.VMEM((1024,), _f32),
            pltpu.SemaphoreType.DMA,
        ],
    )
    out = det(probs2, rois2, bbxt, winflat)
    return out.reshape(_B, 128, 8)[:, :_MAX_INST, :6]
